# baseline (speedup 1.0000x reference)
import functools

import jax
import jax.numpy as jnp
from jax import lax
from jax.experimental import pallas as pl
from jax.experimental.pallas import tpu as pltpu
from jax.experimental.pallas import tpu_sc as plsc

VOCAB = 1_000_000
DIM = 64
BATCH = 16384
LANES = 16
NUM_CORES = 2
NUM_SUBCORES = 16
NUM_WORKERS = NUM_CORES * NUM_SUBCORES
B_PER_W = BATCH // NUM_WORKERS
CHUNK = 128
N_CHUNKS = B_PER_W // CHUNK
BLOCKS = CHUNK // LANES

_LOG1P_COEFS = (
    5.629329962175689e-07,
    0.9999574422836304,
    -0.49920639395713806,
    0.3269723653793335,
    -0.2228347212076187,
    0.13076335191726685,
    -0.05262395367026329,
    0.01011890172958374,
)


def _softplus16(x):
    u = jnp.exp(-jnp.abs(x))
    acc = jnp.full((LANES,), _LOG1P_COEFS[-1], dtype=jnp.float32)
    for c in _LOG1P_COEFS[-2::-1]:
        acc = acc * u + jnp.float32(c)
    return jnp.maximum(x, jnp.float32(0.0)) + acc


def _sc_body(idx_hbm, table_hbm, out_hbm, idx_v, g_v, buf_v, outst_v, sem):
    wid = lax.axis_index("s") * NUM_CORES + lax.axis_index("c")
    base = wid * B_PER_W
    pltpu.sync_copy(idx_hbm.at[pl.ds(base, B_PER_W)], idx_v)

    for t in range(B_PER_W // LANES):
        sl = pl.ds(t * LANES, LANES)
        g_v[sl] = idx_v[sl] >> 1

    iota = lax.iota(jnp.int32, LANES)

    def chunk_body(c, carry):
        pltpu.async_copy(
            table_hbm.at[g_v.at[pl.ds(c * CHUNK, CHUNK)]], buf_v, sem
        ).wait()

        def block_body(b, carry2):
            iv = idx_v[pl.ds(c * CHUNK + b * LANES, LANES)]
            h = jnp.bitwise_and(iv, jnp.int32(1)) * jnp.int32(DIM)
            s = iota + b * LANES
            rows = iota + b * LANES
            for j in range(DIM):
                jf = jnp.full((LANES,), j, dtype=jnp.int32)
                x = plsc.load_gather(buf_v, [s, h + jnp.int32(j)])
                plsc.store_scatter(outst_v, [rows, jf], _softplus16(x))
            return carry2

        lax.fori_loop(0, BLOCKS, block_body, 0)
        pltpu.sync_copy(outst_v, out_hbm.at[pl.ds(base + c * CHUNK, CHUNK)])
        return carry

    lax.fori_loop(0, N_CHUNKS, chunk_body, 0)


def kernel(target_w_id, S):
    idx = target_w_id.astype(jnp.int32)
    table2 = S.reshape(VOCAB // 2, 2 * DIM)
    mesh = plsc.VectorSubcoreMesh(core_axis_name="c", subcore_axis_name="s")
    run = pl.kernel(
        _sc_body,
        mesh=mesh,
        out_type=jax.ShapeDtypeStruct((BATCH, DIM), jnp.float32),
        scratch_types=[
            pltpu.VMEM((B_PER_W,), jnp.int32),
            pltpu.VMEM((B_PER_W,), jnp.int32),
            pltpu.VMEM((CHUNK, 2 * DIM), jnp.float32),
            pltpu.VMEM((CHUNK, DIM), jnp.float32),
            pltpu.SemaphoreType.DMA,
        ],
        compiler_params=pltpu.CompilerParams(needs_layout_passes=False),
    )
    return run(idx, table2)


# speedup vs baseline: 1.1298x; 1.1298x over previous
import functools

import jax
import jax.numpy as jnp
from jax import lax
from jax.experimental import pallas as pl
from jax.experimental.pallas import tpu as pltpu
from jax.experimental.pallas import tpu_sc as plsc

VOCAB = 1_000_000
DIM = 64
BATCH = 16384
LANES = 16
NUM_CORES = 2
NUM_SUBCORES = 16
NUM_WORKERS = NUM_CORES * NUM_SUBCORES
B_PER_W = BATCH // NUM_WORKERS
CHUNK = 128
N_CHUNKS = B_PER_W // CHUNK

_LOG1P_COEFS = (
    5.629329962175689e-07,
    0.9999574422836304,
    -0.49920639395713806,
    0.3269723653793335,
    -0.2228347212076187,
    0.13076335191726685,
    -0.05262395367026329,
    0.01011890172958374,
)


def _softplus16(x):
    u = jnp.exp(-jnp.abs(x))
    acc = jnp.full((LANES,), _LOG1P_COEFS[-1], dtype=jnp.float32)
    for c in _LOG1P_COEFS[-2::-1]:
        acc = acc * u + jnp.float32(c)
    return jnp.maximum(x, jnp.float32(0.0)) + acc


def _sc_body(idx_hbm, table_hbm, out_hbm, idx_v, g_v, buf_v, outst_v, sem):
    wid = lax.axis_index("s") * NUM_CORES + lax.axis_index("c")
    base = wid * B_PER_W
    pltpu.sync_copy(idx_hbm.at[pl.ds(base, B_PER_W)], idx_v)

    for t in range(B_PER_W // LANES):
        sl = pl.ds(t * LANES, LANES)
        g_v[sl] = idx_v[sl] >> 1

    def chunk_body(c, carry):
        pltpu.async_copy(
            table_hbm.at[g_v.at[pl.ds(c * CHUNK, CHUNK)]], buf_v, sem
        ).wait()

        def row_body(r, carry2):
            rsplat = jnp.zeros((LANES,), jnp.int32) + (c * CHUNK + r)
            iv = plsc.load_gather(idx_v, [rsplat])
            hf = jnp.bitwise_and(iv, jnp.int32(1)).astype(jnp.float32)
            for t in range(DIM // LANES):
                x0 = buf_v[r, pl.ds(t * LANES, LANES)]
                x1 = buf_v[r, pl.ds(DIM + t * LANES, LANES)]
                x = x0 + hf * (x1 - x0)
                outst_v[r, pl.ds(t * LANES, LANES)] = _softplus16(x)
            return carry2

        lax.fori_loop(0, CHUNK, row_body, 0)
        pltpu.sync_copy(outst_v, out_hbm.at[pl.ds(base + c * CHUNK, CHUNK)])
        return carry

    lax.fori_loop(0, N_CHUNKS, chunk_body, 0)


def kernel(target_w_id, S):
    idx = target_w_id.astype(jnp.int32)
    table2 = lax.optimization_barrier(S.reshape(VOCAB // 2, 2 * DIM))
    mesh = plsc.VectorSubcoreMesh(core_axis_name="c", subcore_axis_name="s")
    run = pl.kernel(
        _sc_body,
        mesh=mesh,
        out_type=jax.ShapeDtypeStruct((BATCH, DIM), jnp.float32),
        scratch_types=[
            pltpu.VMEM((B_PER_W,), jnp.int32),
            pltpu.VMEM((B_PER_W,), jnp.int32),
            pltpu.VMEM((CHUNK, 2 * DIM), jnp.float32),
            pltpu.VMEM((CHUNK, DIM), jnp.float32),
            pltpu.SemaphoreType.DMA,
        ],
        compiler_params=pltpu.CompilerParams(needs_layout_passes=False),
    )
    return run(idx, table2)


# speedup vs baseline: 1.1306x; 1.0007x over previous
import functools

import jax
import jax.numpy as jnp
from jax import lax
from jax.experimental import pallas as pl
from jax.experimental.pallas import tpu as pltpu
from jax.experimental.pallas import tpu_sc as plsc

VOCAB = 1_000_000
DIM = 64
BATCH = 16384
LANES = 16
NUM_CORES = 2
NUM_SUBCORES = 16
NUM_WORKERS = NUM_CORES * NUM_SUBCORES
B_PER_W = BATCH // NUM_WORKERS
CHUNK = 128
N_CHUNKS = B_PER_W // CHUNK

_LOG1P_COEFS = (
    5.629329962175689e-07,
    0.9999574422836304,
    -0.49920639395713806,
    0.3269723653793335,
    -0.2228347212076187,
    0.13076335191726685,
    -0.05262395367026329,
    0.01011890172958374,
)


def _softplus16(x):
    u = jnp.exp(-jnp.abs(x))
    acc = jnp.full((LANES,), _LOG1P_COEFS[-1], dtype=jnp.float32)
    for c in _LOG1P_COEFS[-2::-1]:
        acc = acc * u + jnp.float32(c)
    return jnp.maximum(x, jnp.float32(0.0)) + acc


def _sc_body(idx_hbm, table_hbm, out_hbm, idx_v, g_v, buf_v, outst_v, sem):
    wid = lax.axis_index("s") * NUM_CORES + lax.axis_index("c")
    base = wid * B_PER_W
    pltpu.sync_copy(idx_hbm.at[pl.ds(base, B_PER_W)], idx_v)

    for t in range(B_PER_W // LANES):
        sl = pl.ds(t * LANES, LANES)
        g_v[sl] = idx_v[sl] >> 1

    def chunk_body(c, carry):
        pltpu.async_copy(
            table_hbm.at[g_v.at[pl.ds(c * CHUNK, CHUNK)]], buf_v, sem
        ).wait()

        def row_body(r, carry2):
            rsplat = jnp.zeros((LANES,), jnp.int32) + (c * CHUNK + r)
            iv = plsc.load_gather(idx_v, [rsplat])
            hf = jnp.bitwise_and(iv, jnp.int32(1)).astype(jnp.float32)
            for t in range(DIM // LANES):
                x0 = buf_v[r, pl.ds(t * LANES, LANES)]
                x1 = buf_v[r, pl.ds(DIM + t * LANES, LANES)]
                x = x0 + hf * (x1 - x0)
                outst_v[r, pl.ds(t * LANES, LANES)] = _softplus16(x)
            return carry2

        lax.fori_loop(0, CHUNK, row_body, 0)
        pltpu.sync_copy(outst_v, out_hbm.at[pl.ds(base + c * CHUNK, CHUNK)])
        return carry

    lax.fori_loop(0, N_CHUNKS, chunk_body, 0)


def kernel(target_w_id, S):
    idx = target_w_id.astype(jnp.int32)
    table2 = lax.optimization_barrier(S.reshape(VOCAB // 2, 2 * DIM))
    mesh = plsc.VectorSubcoreMesh(core_axis_name="c", subcore_axis_name="s")
    run = pl.kernel(
        _sc_body,
        mesh=mesh,
        out_type=jax.ShapeDtypeStruct((BATCH, DIM), jnp.float32),
        scratch_types=[
            pltpu.VMEM((B_PER_W,), jnp.int32),
            pltpu.VMEM((B_PER_W,), jnp.int32),
            pltpu.VMEM((CHUNK, 2 * DIM), jnp.float32),
            pltpu.VMEM((CHUNK, DIM), jnp.float32),
            pltpu.SemaphoreType.DMA,
        ],
        compiler_params=pltpu.CompilerParams(
            needs_layout_passes=False, skip_device_barrier=True
        ),
    )
    return run(idx, table2)
